# baseline (speedup 1.0000x reference)
import jax
import jax.numpy as jnp
from jax import lax
from jax.experimental import pallas as pl
from jax.experimental.pallas import tpu as pltpu
from jax.experimental.pallas import tpu_sc as plsc

VOCAB = 30522
MAX_POS = 2048
HIDDEN = 768
BATCH = 4
SEQ = 2048

NC = 2
NS = 16
NW = NC * NS
LANES = 16
NVEC = HIDDEN // LANES
WPOS = SEQ // NW
CHUNK_P = 8
NCHUNK = WPOS // CHUNK_P
ROWS_C = CHUNK_P * BATCH
WROWS = WPOS * BATCH
NBUF = 4
LOOKAHEAD = NBUF - 2
POS_BLK = 2


def _hsum16(x):
    dnums = lax.GatherDimensionNumbers(
        offset_dims=(), collapsed_slice_dims=(0,), start_index_map=(0,))
    for sh in (8, 4, 2, 1):
        idx = lax.iota(jnp.int32, LANES) ^ sh
        x = x + lax.gather(x, idx[:, None], dnums, (1,),
                           mode=lax.GatherScatterMode.PROMISE_IN_BOUNDS)
    return x


def _rsqrt16(v):
    bits = plsc.bitcast(v, jnp.int32)
    y = plsc.bitcast(jnp.int32(0x5F3759DF) - (bits >> 1), jnp.float32)
    half = v * 0.5
    for _ in range(2):
        y = y * (1.5 - half * y * y)
    return y


def _perm_xor(x, bit):
    dnums = lax.GatherDimensionNumbers(
        offset_dims=(), collapsed_slice_dims=(0,), start_index_map=(0,))
    idx = lax.iota(jnp.int32, LANES) ^ bit
    return lax.gather(x, idx[:, None], dnums, (1,),
                      mode=lax.GatherScatterMode.PROMISE_IN_BOUNDS)


def _tree_sum8(vs):
    lane = lax.iota(jnp.int32, LANES)

    def merge(a, b, bit):
        keep = (lane & bit) == 0
        s = jnp.where(keep, a, b)
        t = jnp.where(keep, b, a)
        return s + _perm_xor(t, bit)

    c = [merge(vs[2 * k], vs[2 * k + 1], 1) for k in range(4)]
    c = [merge(c[2 * k], c[2 * k + 1], 2) for k in range(2)]
    r = merge(c[0], c[1], 4)
    return r + _perm_xor(r, 8)


def _bcast_lane(x, lane):
    dnums = lax.GatherDimensionNumbers(
        offset_dims=(), collapsed_slice_dims=(0,), start_index_map=(0,))
    idx = jnp.full((LANES,), lane, jnp.int32)
    return lax.gather(x, idx[:, None], dnums, (1,),
                      mode=lax.GatherScatterMode.PROMISE_IN_BOUNDS)


def _tec_body(ids_hbm, word_hbm, pos_hbm, gamma_hbm, beta_hbm, out_hbm,
              *scratch):
    idx_all = scratch[0]
    word_v = list(scratch[1:1 + NBUF])
    pos_v = list(scratch[1 + NBUF:1 + 2 * NBUF])
    gamma_v, beta_v = scratch[1 + 2 * NBUF:3 + 2 * NBUF]
    isem = scratch[3 + 2 * NBUF]
    gsem = list(scratch[4 + 2 * NBUF:4 + 3 * NBUF])
    psem = list(scratch[4 + 3 * NBUF:4 + 4 * NBUF])
    osem = list(scratch[4 + 4 * NBUF:4 + 5 * NBUF])

    cid = lax.axis_index("c")
    sid = lax.axis_index("s")
    wid = sid * NC + cid
    pbase = wid * WPOS

    def idx_copy(c, b):
        return pltpu.async_copy(
            ids_hbm.at[pl.ds(b * SEQ + pbase + c * CHUNK_P, CHUNK_P)],
            idx_all.at[pl.ds(c * ROWS_C + b * CHUNK_P, CHUNK_P)], isem)

    ih_first = [idx_copy(c, b)
                for c in range(min(LOOKAHEAD, NCHUNK)) for b in range(BATCH)]

    ghandle = [None] * NBUF
    phandle = [None] * NBUF
    ohandle = [None] * NBUF

    def stage(c):
        buf = c % NBUF
        ghandle[buf] = pltpu.async_copy(
            word_hbm.at[idx_all.at[pl.ds(c * ROWS_C, ROWS_C)]],
            word_v[buf], gsem[buf])
        phandle[buf] = pltpu.async_copy(
            pos_hbm.at[pl.ds(pbase + c * CHUNK_P, CHUNK_P)],
            pos_v[buf], psem[buf])

    for h in ih_first:
        h.wait()
    for s in range(min(LOOKAHEAD, NCHUNK)):
        stage(s)
    ih_rest = [idx_copy(c, b)
               for c in range(min(LOOKAHEAD, NCHUNK), NCHUNK)
               for b in range(BATCH)]
    pltpu.sync_copy(gamma_hbm, gamma_v)
    pltpu.sync_copy(beta_hbm, beta_v)
    for h in ih_rest:
        h.wait()

    for c in range(NCHUNK):
        buf = c % NBUF
        s = c + LOOKAHEAD
        if s < NCHUNK:
            sbuf = s % NBUF
            if ohandle[sbuf] is not None:
                for h in ohandle[sbuf]:
                    h.wait()
                ohandle[sbuf] = None
            stage(s)
        ghandle[buf].wait()
        phandle[buf].wait()
        pb = pbase + c * CHUNK_P
        word_c = word_v[buf]
        pos_c = pos_v[buf]

        def pos_body(i2, _, word_v=word_c, pos_v=pos_c):
            i = i2 * POS_BLK
            nrow = POS_BLK * BATCH
            rows = [(pi, b) for pi in range(POS_BLK) for b in range(BATCH)]
            zero = jnp.zeros((LANES,), jnp.float32)
            init = tuple([zero] * (2 * nrow))

            def sum_body(j, carry):
                a = list(carry[:nrow])
                q = list(carry[nrow:])
                sl = pl.ds(j * LANES, LANES)
                p = [pos_v[i + pi, sl] for pi in range(POS_BLK)]
                for r, (pi, b) in enumerate(rows):
                    x = word_v[b * CHUNK_P + i + pi, sl] + p[pi]
                    word_v[b * CHUNK_P + i + pi, sl] = x
                    a[r] = a[r] + x
                    q[r] = q[r] + x * x
                return tuple(a) + tuple(q)

            carry = plsc.parallel_loop(0, NVEC, unroll=3, carry=init)(sum_body)
            mean_p = _tree_sum8(list(carry[:nrow])) * (1.0 / HIDDEN)
            s2_p = _tree_sum8(list(carry[nrow:])) * (1.0 / HIDDEN)
            var_p = s2_p - mean_p * mean_p
            rstd_p = _rsqrt16(var_p + 1e-12)
            mean = [_bcast_lane(mean_p, r) for r in range(nrow)]
            rstd = [_bcast_lane(rstd_p, r) for r in range(nrow)]

            def norm_body(j):
                sl = pl.ds(j * LANES, LANES)
                g = gamma_v[sl]
                bt = beta_v[sl]
                for r, (pi, b) in enumerate(rows):
                    x = word_v[b * CHUNK_P + i + pi, sl]
                    word_v[b * CHUNK_P + i + pi, sl] = \
                        (x - mean[r]) * rstd[r] * g + bt

            plsc.parallel_loop(0, NVEC, unroll=3)(norm_body)
            return _

        lax.fori_loop(0, CHUNK_P // POS_BLK, pos_body, None)

        ohandle[buf] = [
            pltpu.async_copy(word_c.at[pl.ds(b * CHUNK_P, CHUNK_P)],
                             out_hbm.at[pl.ds(b * SEQ + pb, CHUNK_P)],
                             osem[buf])
            for b in range(BATCH)
        ]

    for hs in ohandle:
        if hs is not None:
            for h in hs:
                h.wait()


def kernel(input_ids, word_table, pos_table, gamma, beta):
    ids_flat = input_ids.reshape(-1).astype(jnp.int32)
    mesh = plsc.VectorSubcoreMesh(core_axis_name="c", subcore_axis_name="s")
    scratch = (
        [pltpu.VMEM((WROWS,), jnp.int32)]
        + [pltpu.VMEM((ROWS_C, HIDDEN), jnp.float32)] * NBUF
        + [pltpu.VMEM((CHUNK_P, HIDDEN), jnp.float32)] * NBUF
        + [pltpu.VMEM((HIDDEN,), jnp.float32)] * 2
        + [pltpu.SemaphoreType.DMA] * (1 + 3 * NBUF)
    )
    call = pl.kernel(
        _tec_body,
        mesh=mesh,
        out_type=jax.ShapeDtypeStruct((BATCH * SEQ, HIDDEN), jnp.float32),
        scratch_types=scratch,
        compiler_params=pltpu.CompilerParams(needs_layout_passes=False),
    )
    out = call(ids_flat, word_table, pos_table, gamma, beta)
    return out.reshape(BATCH, SEQ, HIDDEN)


# speedup vs baseline: 1.1433x; 1.1433x over previous
import jax
import jax.numpy as jnp
from jax import lax
from jax.experimental import pallas as pl
from jax.experimental.pallas import tpu as pltpu
from jax.experimental.pallas import tpu_sc as plsc

VOCAB = 30522
MAX_POS = 2048
HIDDEN = 768
BATCH = 4
SEQ = 2048

NC = 2
NS = 16
NW = NC * NS
LANES = 16
NVEC = HIDDEN // LANES
WPOS = SEQ // NW
CHUNK_P = 8
NCHUNK = WPOS // CHUNK_P
ROWS_C = CHUNK_P * BATCH
WROWS = WPOS * BATCH
NBUF = 4
LOOKAHEAD = NBUF - 2
POS_BLK = 2


def _hsum16(x):
    dnums = lax.GatherDimensionNumbers(
        offset_dims=(), collapsed_slice_dims=(0,), start_index_map=(0,))
    for sh in (8, 4, 2, 1):
        idx = lax.iota(jnp.int32, LANES) ^ sh
        x = x + lax.gather(x, idx[:, None], dnums, (1,),
                           mode=lax.GatherScatterMode.PROMISE_IN_BOUNDS)
    return x


def _rsqrt16(v):
    bits = plsc.bitcast(v, jnp.int32)
    y = plsc.bitcast(jnp.int32(0x5F3759DF) - (bits >> 1), jnp.float32)
    half = v * 0.5
    for _ in range(2):
        y = y * (1.5 - half * y * y)
    return y


def _perm_xor(x, bit):
    dnums = lax.GatherDimensionNumbers(
        offset_dims=(), collapsed_slice_dims=(0,), start_index_map=(0,))
    idx = lax.iota(jnp.int32, LANES) ^ bit
    return lax.gather(x, idx[:, None], dnums, (1,),
                      mode=lax.GatherScatterMode.PROMISE_IN_BOUNDS)


def _tree_sum8(vs):
    lane = lax.iota(jnp.int32, LANES)

    def merge(a, b, bit):
        keep = (lane & bit) == 0
        s = jnp.where(keep, a, b)
        t = jnp.where(keep, b, a)
        return s + _perm_xor(t, bit)

    c = [merge(vs[2 * k], vs[2 * k + 1], 1) for k in range(4)]
    c = [merge(c[2 * k], c[2 * k + 1], 2) for k in range(2)]
    r = merge(c[0], c[1], 4)
    return r + _perm_xor(r, 8)


def _bcast_lane(x, lane):
    dnums = lax.GatherDimensionNumbers(
        offset_dims=(), collapsed_slice_dims=(0,), start_index_map=(0,))
    idx = jnp.full((LANES,), lane, jnp.int32)
    return lax.gather(x, idx[:, None], dnums, (1,),
                      mode=lax.GatherScatterMode.PROMISE_IN_BOUNDS)


def _tec_body(ids_hbm, word_hbm, pos_hbm, gamma_hbm, beta_hbm, out_hbm,
              *scratch):
    idx_all = scratch[0]
    word_v = list(scratch[1:1 + NBUF])
    pos_v = list(scratch[1 + NBUF:1 + 2 * NBUF])
    gamma_v, beta_v = scratch[1 + 2 * NBUF:3 + 2 * NBUF]
    isem = scratch[3 + 2 * NBUF]
    gsem = list(scratch[4 + 2 * NBUF:4 + 3 * NBUF])
    psem = list(scratch[4 + 3 * NBUF:4 + 4 * NBUF])
    osem = list(scratch[4 + 4 * NBUF:4 + 5 * NBUF])

    cid = lax.axis_index("c")
    sid = lax.axis_index("s")
    wid = sid * NC + cid
    pbase = wid * WPOS

    def idx_copy(c, b):
        return pltpu.async_copy(
            ids_hbm.at[pl.ds(b * SEQ + pbase + c * CHUNK_P, CHUNK_P)],
            idx_all.at[pl.ds(c * ROWS_C + b * CHUNK_P, CHUNK_P)], isem)

    ih_first = [idx_copy(c, b)
                for c in range(min(LOOKAHEAD, NCHUNK)) for b in range(BATCH)]

    ghandle = [None] * NBUF
    phandle = [None] * NBUF
    ohandle = [None] * NBUF

    def stage(c):
        buf = c % NBUF
        ghandle[buf] = pltpu.async_copy(
            word_hbm.at[idx_all.at[pl.ds(c * ROWS_C, ROWS_C)]],
            word_v[buf], gsem[buf])
        phandle[buf] = pltpu.async_copy(
            pos_hbm.at[pl.ds(pbase + c * CHUNK_P, CHUNK_P)],
            pos_v[buf], psem[buf])

    for h in ih_first:
        h.wait()
    for s in range(min(LOOKAHEAD, NCHUNK)):
        stage(s)
    ih_rest = [idx_copy(c, b)
               for c in range(min(LOOKAHEAD, NCHUNK), NCHUNK)
               for b in range(BATCH)]
    pltpu.sync_copy(gamma_hbm, gamma_v)
    pltpu.sync_copy(beta_hbm, beta_v)
    for h in ih_rest:
        h.wait()

    for c in range(NCHUNK):
        buf = c % NBUF
        s = c + LOOKAHEAD
        if s < NCHUNK:
            sbuf = s % NBUF
            if ohandle[sbuf] is not None:
                for h in ohandle[sbuf]:
                    h.wait()
                ohandle[sbuf] = None
            stage(s)
        ghandle[buf].wait()
        phandle[buf].wait()
        pb = pbase + c * CHUNK_P
        word_c = word_v[buf]
        pos_c = pos_v[buf]

        def pos_body(i2, _, word_v=word_c, pos_v=pos_c):
            i = i2 * POS_BLK
            nrow = POS_BLK * BATCH
            rows = [(pi, b) for pi in range(POS_BLK) for b in range(BATCH)]
            zero = jnp.zeros((LANES,), jnp.float32)
            init = tuple([zero] * (2 * nrow))

            def sum_body(j, carry):
                a = list(carry[:nrow])
                q = list(carry[nrow:])
                sl = pl.ds(j * LANES, LANES)
                p = [pos_v[i + pi, sl] for pi in range(POS_BLK)]
                for r, (pi, b) in enumerate(rows):
                    x = word_v[b * CHUNK_P + i + pi, sl] + p[pi]
                    word_v[b * CHUNK_P + i + pi, sl] = x
                    a[r] = a[r] + x
                    q[r] = q[r] + x * x
                return tuple(a) + tuple(q)

            carry = plsc.parallel_loop(0, NVEC, unroll=2, carry=init)(sum_body)
            mean_p = _tree_sum8(list(carry[:nrow])) * (1.0 / HIDDEN)
            s2_p = _tree_sum8(list(carry[nrow:])) * (1.0 / HIDDEN)
            var_p = s2_p - mean_p * mean_p
            rstd_p = _rsqrt16(var_p + 1e-12)
            mean = [_bcast_lane(mean_p, r) for r in range(nrow)]
            rstd = [_bcast_lane(rstd_p, r) for r in range(nrow)]

            def norm_body(j):
                sl = pl.ds(j * LANES, LANES)
                g = gamma_v[sl]
                bt = beta_v[sl]
                for r, (pi, b) in enumerate(rows):
                    x = word_v[b * CHUNK_P + i + pi, sl]
                    word_v[b * CHUNK_P + i + pi, sl] = \
                        (x - mean[r]) * rstd[r] * g + bt

            plsc.parallel_loop(0, NVEC, unroll=2)(norm_body)
            return _

        lax.fori_loop(0, CHUNK_P // POS_BLK, pos_body, None)

        ohandle[buf] = [
            pltpu.async_copy(word_c.at[pl.ds(b * CHUNK_P, CHUNK_P)],
                             out_hbm.at[pl.ds(b * SEQ + pb, CHUNK_P)],
                             osem[buf])
            for b in range(BATCH)
        ]

    for hs in ohandle:
        if hs is not None:
            for h in hs:
                h.wait()


def kernel(input_ids, word_table, pos_table, gamma, beta):
    ids_flat = input_ids.reshape(-1).astype(jnp.int32)
    mesh = plsc.VectorSubcoreMesh(core_axis_name="c", subcore_axis_name="s")
    scratch = (
        [pltpu.VMEM((WROWS,), jnp.int32)]
        + [pltpu.VMEM((ROWS_C, HIDDEN), jnp.float32)] * NBUF
        + [pltpu.VMEM((CHUNK_P, HIDDEN), jnp.float32)] * NBUF
        + [pltpu.VMEM((HIDDEN,), jnp.float32)] * 2
        + [pltpu.SemaphoreType.DMA] * (1 + 3 * NBUF)
    )
    call = pl.kernel(
        _tec_body,
        mesh=mesh,
        out_type=jax.ShapeDtypeStruct((BATCH * SEQ, HIDDEN), jnp.float32),
        scratch_types=scratch,
        compiler_params=pltpu.CompilerParams(needs_layout_passes=False),
    )
    out = call(ids_flat, word_table, pos_table, gamma, beta)
    return out.reshape(BATCH, SEQ, HIDDEN)
